# Initial kernel scaffold; baseline (speedup 1.0000x reference)
import jax
import jax.numpy as jnp
from jax.experimental import pallas as pl


def kernel(x, w_score, bin_prob_logits, Wv):
    raise NotImplementedError("write your pallas kernel here")


# speedup vs baseline: 9.3913x; 9.3913x over previous
import functools

import jax
import jax.numpy as jnp
from jax.experimental import pallas as pl

_NUM_BINS = 6
_M = 8192


def _waterfill(total, bin_prob, max_num_points):
    Bv, num_bins = bin_prob.shape
    bp = bin_prob * max_num_points + 1e-10
    chosen = jnp.zeros_like(bp)
    for _ in range(num_bins):
        bp = bp / jnp.sum(bp, axis=1, keepdims=True)
        num_to_choose = total - jnp.sum(chosen, axis=1, keepdims=True)
        chosen = chosen + bp * num_to_choose
        chosen = jnp.where(chosen >= max_num_points, max_num_points, chosen)
        bp = bp * jnp.where(chosen >= max_num_points, 0.0, 1.0)
    chosen = chosen.astype(jnp.int32)
    deficit = total - jnp.sum(chosen, axis=1)
    arg = jnp.argmax(max_num_points.astype(jnp.int32) - chosen, axis=1)
    chosen = chosen.at[jnp.arange(Bv), arg].add(deficit.astype(chosen.dtype))
    return chosen


def _desc_key(s):
    b = jax.lax.bitcast_convert_type(s, jnp.int32)
    u = jnp.where(b < 0, ~b, b ^ jnp.int32(-2147483648)).astype(jnp.uint32)
    return ~u


def _build_rows(s, k_choose, bvals, Mv):
    Bv, Nv = s.shape
    binid = jnp.sum(s[:, :, None] < bvals[None, None, :], axis=2).astype(jnp.int32)
    onehot = binid[:, :, None] == jnp.arange(_NUM_BINS)[None, None, :]
    counts = jnp.sum(onehot, axis=1).astype(jnp.int32)
    q = s + jnp.float32(1e-08)
    offsets = jnp.cumsum(k_choose, axis=1) - k_choose
    Cj = jnp.cumsum(counts, axis=1) - counts
    srt = jnp.argsort(_desc_key(s), axis=1, stable=True).astype(jnp.int32)
    q_sorted = jnp.take_along_axis(q, srt, axis=1)
    bin_sorted = jnp.take_along_axis(binid, srt, axis=1)

    P = jnp.sum(onehot & (q > 0)[:, :, None], axis=1).astype(jnp.int32)
    Z = jnp.sum(onehot & (q == 0)[:, :, None], axis=1).astype(jnp.int32)
    G23 = (Nv - counts) + Z
    t = jnp.minimum(k_choose, P)
    r2 = jnp.maximum(k_choose - t, 0)
    c23 = jnp.minimum(r2, G23)
    r4 = jnp.maximum(r2 - G23, 0)

    def takej(a, jj):
        return jnp.take_along_axis(a, jj, axis=1)

    pos = jnp.arange(Nv, dtype=jnp.int32)[None, :]
    jj = bin_sorted
    w = pos - takej(Cj, jj)
    sel1 = (q_sorted > 0) & (w < takej(t, jj))
    dest1 = takej(offsets, jj) + w
    g4pos = w - takej(P, jj) - takej(Z, jj)
    sel4 = (q_sorted < 0) & (g4pos >= 0) & (g4pos < takej(r4, jj))
    dest4 = takej(offsets, jj) + takej(t, jj) + takej(c23, jj) + g4pos
    dest = jnp.where(sel1, dest1, jnp.where(sel4, dest4, Mv))
    barange = jnp.arange(Bv)[:, None]
    rows = jnp.zeros((Bv, Mv), jnp.int32)
    rows = rows.at[barange, dest].set(srt, mode='drop')

    def cumexc(m):
        c = jnp.cumsum(m.astype(jnp.int32), axis=1)
        return c - m.astype(jnp.int32)

    ids = jnp.broadcast_to(jnp.arange(Nv, dtype=jnp.int32)[None, :], (Bv, Nv))
    for j in range(_NUM_BINS):
        ind23 = (binid != j) | (q == 0)
        pos23 = cumexc(ind23)
        sel2 = ind23 & (pos23 < r2[:, j, None])
        dest2 = offsets[:, j, None] + t[:, j, None] + pos23
        rows = rows.at[barange, jnp.where(sel2, dest2, Mv)].set(ids, mode='drop')
    return rows


def _mm_body(wv_ref, xd_ref, o_ref):
    o_ref[0] = jax.lax.dot(
        wv_ref[...], xd_ref[0], preferred_element_type=jnp.float32)


def _project(Wv, x_down):
    Bv, Cv, Mv = x_down.shape
    BM = 2048
    return pl.pallas_call(
        _mm_body,
        grid=(Bv, Mv // BM),
        in_specs=[
            pl.BlockSpec((Cv, Cv), lambda b, m: (0, 0)),
            pl.BlockSpec((1, Cv, BM), lambda b, m: (b, 0, m)),
        ],
        out_specs=pl.BlockSpec((1, Cv, BM), lambda b, m: (b, 0, m)),
        out_shape=jax.ShapeDtypeStruct((Bv, Cv, Mv), jnp.float32),
    )(Wv, x_down)


def kernel(x, w_score, bin_prob_logits, Wv):
    Bv, Cv, Nv = x.shape
    score = jnp.einsum('bcn,c->bn', x, w_score)[:, None, :]
    s4 = (score - jnp.mean(score, axis=2, keepdims=True)) / jnp.std(
        score, axis=2, keepdims=True) + 1e-08
    s = s4.reshape(Bv, Nv)
    n = Bv * Nv
    bidx = (jnp.arange(1, _NUM_BINS) / _NUM_BINS * n).astype(jnp.int32)
    sorted_scores = jnp.sort(s.reshape(-1))[::-1]
    bvals = sorted_scores[bidx]
    binid = jnp.sum(s[:, :, None] < bvals[None, None, :], axis=2)
    counts = jnp.sum(binid[:, :, None] == jnp.arange(_NUM_BINS)[None, None, :],
                     axis=1)
    bin_prob = jnp.broadcast_to(jax.nn.softmax(bin_prob_logits)[None, :],
                                (Bv, _NUM_BINS))
    k_choose = _waterfill(_M, bin_prob, counts.astype(jnp.float32))
    rows = _build_rows(s, k_choose, bvals, _M)
    index_down = rows.reshape(Bv, 1, _M)
    idx = jnp.broadcast_to(index_down, (Bv, Cv, _M))
    x_down = jnp.take_along_axis(x, idx, axis=2)
    out = _project(Wv, x_down)
    return out, index_down


# speedup vs baseline: 12.7863x; 1.3615x over previous
import functools

import jax
import jax.numpy as jnp
from jax.experimental import pallas as pl
from jax.experimental.pallas import tpu as pltpu
from jax.experimental.pallas import tpu_sc as plsc

_NUM_BINS = 6
_M = 8192
_HALF = _M // 2
_CH = 8192


def _waterfill(total, bin_prob, max_num_points):
    Bv, num_bins = bin_prob.shape
    bp = bin_prob * max_num_points + 1e-10
    chosen = jnp.zeros_like(bp)
    for _ in range(num_bins):
        bp = bp / jnp.sum(bp, axis=1, keepdims=True)
        num_to_choose = total - jnp.sum(chosen, axis=1, keepdims=True)
        chosen = chosen + bp * num_to_choose
        chosen = jnp.where(chosen >= max_num_points, max_num_points, chosen)
        bp = bp * jnp.where(chosen >= max_num_points, 0.0, 1.0)
    chosen = chosen.astype(jnp.int32)
    deficit = total - jnp.sum(chosen, axis=1)
    arg = jnp.argmax(max_num_points.astype(jnp.int32) - chosen, axis=1)
    chosen = chosen.at[jnp.arange(Bv), arg].add(deficit.astype(chosen.dtype))
    return chosen


def _desc_key(s):
    b = jax.lax.bitcast_convert_type(s, jnp.int32)
    u = jnp.where(b < 0, ~b, b ^ jnp.int32(-2147483648)).astype(jnp.uint32)
    return ~u


def _build_rows(s, k_choose, bvals, Mv):
    Bv, Nv = s.shape
    binid = jnp.sum(s[:, :, None] < bvals[None, None, :], axis=2).astype(jnp.int32)
    onehot = binid[:, :, None] == jnp.arange(_NUM_BINS)[None, None, :]
    counts = jnp.sum(onehot, axis=1).astype(jnp.int32)
    q = s + jnp.float32(1e-08)
    offsets = jnp.cumsum(k_choose, axis=1) - k_choose
    Cj = jnp.cumsum(counts, axis=1) - counts
    srt = jnp.argsort(_desc_key(s), axis=1, stable=True).astype(jnp.int32)
    q_sorted = jnp.take_along_axis(q, srt, axis=1)
    bin_sorted = jnp.take_along_axis(binid, srt, axis=1)

    P = jnp.sum(onehot & (q > 0)[:, :, None], axis=1).astype(jnp.int32)
    Z = jnp.sum(onehot & (q == 0)[:, :, None], axis=1).astype(jnp.int32)
    G23 = (Nv - counts) + Z
    t = jnp.minimum(k_choose, P)
    r2 = jnp.maximum(k_choose - t, 0)
    c23 = jnp.minimum(r2, G23)
    r4 = jnp.maximum(r2 - G23, 0)

    def takej(a, jj):
        return jnp.take_along_axis(a, jj, axis=1)

    pos = jnp.arange(Nv, dtype=jnp.int32)[None, :]
    jj = bin_sorted
    w = pos - takej(Cj, jj)
    sel1 = (q_sorted > 0) & (w < takej(t, jj))
    dest1 = takej(offsets, jj) + w
    g4pos = w - takej(P, jj) - takej(Z, jj)
    sel4 = (q_sorted < 0) & (g4pos >= 0) & (g4pos < takej(r4, jj))
    dest4 = takej(offsets, jj) + takej(t, jj) + takej(c23, jj) + g4pos
    destA = jnp.where(sel1, dest1, jnp.where(sel4, dest4, Mv))

    def cumexc(m):
        c = jnp.cumsum(m.astype(jnp.int32), axis=1)
        return c - m.astype(jnp.int32)

    destB = []
    for j in range(_NUM_BINS):
        ind23 = (binid != j) | (q == 0)
        pos23 = cumexc(ind23)
        sel2 = ind23 & (pos23 < r2[:, j, None])
        dest2 = offsets[:, j, None] + t[:, j, None] + pos23
        destB.append(jnp.where(sel2, dest2, Mv))
    return _sc_scatter_rows(destA, srt, jnp.stack(destB, axis=0))


def _sc_rows_body(destA_hbm, srtv_hbm, destB_hbm, rows_hbm, dbuf, vbuf, rbuf):
    cid = jax.lax.axis_index("c")
    sid = jax.lax.axis_index("s")
    w = sid * 2 + cid
    b = w // 2
    hbase = (w % 2) * _HALF
    nv = _CH // 16

    zero = jnp.zeros((16,), jnp.int32)

    def zinit(i, _):
        rbuf[pl.ds(i * 16, 16)] = zero
        return 0

    jax.lax.fori_loop(0, _HALF // 16, zinit, 0)

    lane = jax.lax.iota(jnp.int32, 16)
    nchunks = destA_hbm.shape[1] // _CH

    def scatter_vec(d, v):
        m = (d >= 0) & (d < _HALF)
        dc = jnp.clip(d, 0, _HALF - 1)
        plsc.store_scatter(rbuf, [dc], v, mask=m)

    def passA(c, _):
        pltpu.sync_copy(destA_hbm.at[b, pl.ds(c * _CH, _CH)], dbuf)
        pltpu.sync_copy(srtv_hbm.at[b, pl.ds(c * _CH, _CH)], vbuf)

        def inner(i, _):
            scatter_vec(dbuf[pl.ds(i * 16, 16)] - hbase,
                        vbuf[pl.ds(i * 16, 16)])
            return 0

        jax.lax.fori_loop(0, nv, inner, 0)
        return 0

    jax.lax.fori_loop(0, nchunks, passA, 0)

    for j in range(_NUM_BINS):
        def passB(c, _, j=j):
            pltpu.sync_copy(destB_hbm.at[j, b, pl.ds(c * _CH, _CH)], dbuf)

            def inner(i, _):
                v = c * _CH + i * 16 + lane
                scatter_vec(dbuf[pl.ds(i * 16, 16)] - hbase, v)
                return 0

            jax.lax.fori_loop(0, nv, inner, 0)
            return 0

        jax.lax.fori_loop(0, nchunks, passB, 0)

    pltpu.sync_copy(rbuf, rows_hbm.at[b, pl.ds(hbase, _HALF)])


def _sc_scatter_rows(destA, srtv, destB):
    Bv = destA.shape[0]
    mesh = plsc.VectorSubcoreMesh(core_axis_name="c", subcore_axis_name="s")
    f = functools.partial(
        pl.kernel,
        out_type=jax.ShapeDtypeStruct((Bv, _M), jnp.int32),
        mesh=mesh,
        scratch_types=[
            pltpu.VMEM((_CH,), jnp.int32),
            pltpu.VMEM((_CH,), jnp.int32),
            pltpu.VMEM((_HALF,), jnp.int32),
        ],
        compiler_params=pltpu.CompilerParams(needs_layout_passes=False),
    )(_sc_rows_body)
    return f(destA, srtv, destB)


def _mm_body(wv_ref, xd_ref, o_ref):
    o_ref[0] = jax.lax.dot(
        wv_ref[...], xd_ref[0], preferred_element_type=jnp.float32)


def _project(Wv, x_down):
    Bv, Cv, Mv = x_down.shape
    BM = 2048
    return pl.pallas_call(
        _mm_body,
        grid=(Bv, Mv // BM),
        in_specs=[
            pl.BlockSpec((Cv, Cv), lambda b, m: (0, 0)),
            pl.BlockSpec((1, Cv, BM), lambda b, m: (b, 0, m)),
        ],
        out_specs=pl.BlockSpec((1, Cv, BM), lambda b, m: (b, 0, m)),
        out_shape=jax.ShapeDtypeStruct((Bv, Cv, Mv), jnp.float32),
    )(Wv, x_down)


def kernel(x, w_score, bin_prob_logits, Wv):
    Bv, Cv, Nv = x.shape
    score = jnp.einsum('bcn,c->bn', x, w_score)[:, None, :]
    s4 = (score - jnp.mean(score, axis=2, keepdims=True)) / jnp.std(
        score, axis=2, keepdims=True) + 1e-08
    s = s4.reshape(Bv, Nv)
    n = Bv * Nv
    bidx = (jnp.arange(1, _NUM_BINS) / _NUM_BINS * n).astype(jnp.int32)
    sorted_scores = jnp.sort(s.reshape(-1))[::-1]
    bvals = sorted_scores[bidx]
    binid = jnp.sum(s[:, :, None] < bvals[None, None, :], axis=2)
    counts = jnp.sum(binid[:, :, None] == jnp.arange(_NUM_BINS)[None, None, :],
                     axis=1)
    bin_prob = jnp.broadcast_to(jax.nn.softmax(bin_prob_logits)[None, :],
                                (Bv, _NUM_BINS))
    k_choose = _waterfill(_M, bin_prob, counts.astype(jnp.float32))
    rows = _build_rows(s, k_choose, bvals, _M)
    index_down = rows.reshape(Bv, 1, _M)
    idx = jnp.broadcast_to(index_down, (Bv, Cv, _M))
    x_down = jnp.take_along_axis(x, idx, axis=2)
    out = _project(Wv, x_down)
    return out, index_down


# speedup vs baseline: 15.4293x; 1.2067x over previous
import functools

import jax
import jax.numpy as jnp
from jax.experimental import pallas as pl
from jax.experimental.pallas import tpu as pltpu
from jax.experimental.pallas import tpu_sc as plsc

_NUM_BINS = 6
_M = 8192
_HALF = _M // 2
_CH = 8192


def _waterfill(total, bin_prob, max_num_points):
    Bv, num_bins = bin_prob.shape
    bp = bin_prob * max_num_points + 1e-10
    chosen = jnp.zeros_like(bp)
    for _ in range(num_bins):
        bp = bp / jnp.sum(bp, axis=1, keepdims=True)
        num_to_choose = total - jnp.sum(chosen, axis=1, keepdims=True)
        chosen = chosen + bp * num_to_choose
        chosen = jnp.where(chosen >= max_num_points, max_num_points, chosen)
        bp = bp * jnp.where(chosen >= max_num_points, 0.0, 1.0)
    chosen = chosen.astype(jnp.int32)
    deficit = total - jnp.sum(chosen, axis=1)
    arg = jnp.argmax(max_num_points.astype(jnp.int32) - chosen, axis=1)
    chosen = chosen.at[jnp.arange(Bv), arg].add(deficit.astype(chosen.dtype))
    return chosen


def _desc_key(s):
    b = jax.lax.bitcast_convert_type(s, jnp.int32)
    u = jnp.where(b < 0, ~b, b ^ jnp.int32(-2147483648)).astype(jnp.uint32)
    return ~u


def _build_rows(s, k_choose, bvals, Mv):
    Bv, Nv = s.shape
    binid = jnp.sum(s[:, :, None] < bvals[None, None, :], axis=2).astype(jnp.int32)
    onehot = binid[:, :, None] == jnp.arange(_NUM_BINS)[None, None, :]
    counts = jnp.sum(onehot, axis=1).astype(jnp.int32)
    q = s + jnp.float32(1e-08)
    offsets = jnp.cumsum(k_choose, axis=1) - k_choose
    Cj = jnp.cumsum(counts, axis=1) - counts
    srt = jnp.argsort(_desc_key(s), axis=1, stable=True).astype(jnp.int32)
    q_sorted = jnp.take_along_axis(q, srt, axis=1)
    bin_sorted = jnp.take_along_axis(binid, srt, axis=1)

    P = jnp.sum(onehot & (q > 0)[:, :, None], axis=1).astype(jnp.int32)
    Z = jnp.sum(onehot & (q == 0)[:, :, None], axis=1).astype(jnp.int32)
    G23 = (Nv - counts) + Z
    t = jnp.minimum(k_choose, P)
    r2 = jnp.maximum(k_choose - t, 0)
    c23 = jnp.minimum(r2, G23)
    r4 = jnp.maximum(r2 - G23, 0)

    def takej(a, jj):
        return jnp.take_along_axis(a, jj, axis=1)

    pos = jnp.arange(Nv, dtype=jnp.int32)[None, :]
    jj = bin_sorted
    w = pos - takej(Cj, jj)
    sel1 = (q_sorted > 0) & (w < takej(t, jj))
    dest1 = takej(offsets, jj) + w
    g4pos = w - takej(P, jj) - takej(Z, jj)
    sel4 = (q_sorted < 0) & (g4pos >= 0) & (g4pos < takej(r4, jj))
    dest4 = takej(offsets, jj) + takej(t, jj) + takej(c23, jj) + g4pos
    destA = jnp.where(sel1, dest1, jnp.where(sel4, dest4, Mv))

    ind23 = (binid[None, :, :] != jnp.arange(_NUM_BINS)[:, None, None]) | \
        (q == 0)[None, :, :]
    c = ind23.astype(jnp.int32)
    k = 1
    while k < Nv:
        c = c + jnp.concatenate(
            [jnp.zeros_like(c[..., :k]), c[..., :-k]], axis=-1)
        k *= 2
    pos23 = c - ind23.astype(jnp.int32)
    sel2 = ind23 & (pos23 < r2.T[:, :, None])
    dest2 = offsets.T[:, :, None] + t.T[:, :, None] + pos23
    destB = jnp.where(sel2, dest2, Mv)
    return _sc_scatter_rows(destA, srt, destB)


def _sc_rows_body(destA_hbm, srtv_hbm, destB_hbm, rows_hbm, dbuf, vbuf, rbuf):
    cid = jax.lax.axis_index("c")
    sid = jax.lax.axis_index("s")
    w = sid * 2 + cid
    b = w // 2
    hbase = (w % 2) * _HALF
    nv = _CH // 16

    zero = jnp.zeros((16,), jnp.int32)

    def zinit(i, _):
        rbuf[pl.ds(i * 16, 16)] = zero
        return 0

    jax.lax.fori_loop(0, _HALF // 16, zinit, 0)

    lane = jax.lax.iota(jnp.int32, 16)
    nchunks = destA_hbm.shape[1] // _CH

    def scatter_vec(d, v):
        m = (d >= 0) & (d < _HALF)
        dc = jnp.clip(d, 0, _HALF - 1)
        plsc.store_scatter(rbuf, [dc], v, mask=m)

    def passA(c, _):
        pltpu.sync_copy(destA_hbm.at[b, pl.ds(c * _CH, _CH)], dbuf)
        pltpu.sync_copy(srtv_hbm.at[b, pl.ds(c * _CH, _CH)], vbuf)

        def inner(i, _):
            scatter_vec(dbuf[pl.ds(i * 16, 16)] - hbase,
                        vbuf[pl.ds(i * 16, 16)])
            return 0

        jax.lax.fori_loop(0, nv, inner, 0)
        return 0

    jax.lax.fori_loop(0, nchunks, passA, 0)

    for j in range(_NUM_BINS):
        def passB(c, _, j=j):
            pltpu.sync_copy(destB_hbm.at[j, b, pl.ds(c * _CH, _CH)], dbuf)

            def inner(i, _):
                v = c * _CH + i * 16 + lane
                scatter_vec(dbuf[pl.ds(i * 16, 16)] - hbase, v)
                return 0

            jax.lax.fori_loop(0, nv, inner, 0)
            return 0

        jax.lax.fori_loop(0, nchunks, passB, 0)

    pltpu.sync_copy(rbuf, rows_hbm.at[b, pl.ds(hbase, _HALF)])


def _sc_scatter_rows(destA, srtv, destB):
    Bv = destA.shape[0]
    mesh = plsc.VectorSubcoreMesh(core_axis_name="c", subcore_axis_name="s")
    f = functools.partial(
        pl.kernel,
        out_type=jax.ShapeDtypeStruct((Bv, _M), jnp.int32),
        mesh=mesh,
        scratch_types=[
            pltpu.VMEM((_CH,), jnp.int32),
            pltpu.VMEM((_CH,), jnp.int32),
            pltpu.VMEM((_HALF,), jnp.int32),
        ],
        compiler_params=pltpu.CompilerParams(needs_layout_passes=False),
    )(_sc_rows_body)
    return f(destA, srtv, destB)


def _mm_body(wv_ref, xd_ref, o_ref):
    o_ref[0] = jax.lax.dot(
        wv_ref[...], xd_ref[0], preferred_element_type=jnp.float32)


def _project(Wv, x_down):
    Bv, Cv, Mv = x_down.shape
    BM = 2048
    return pl.pallas_call(
        _mm_body,
        grid=(Bv, Mv // BM),
        in_specs=[
            pl.BlockSpec((Cv, Cv), lambda b, m: (0, 0)),
            pl.BlockSpec((1, Cv, BM), lambda b, m: (b, 0, m)),
        ],
        out_specs=pl.BlockSpec((1, Cv, BM), lambda b, m: (b, 0, m)),
        out_shape=jax.ShapeDtypeStruct((Bv, Cv, Mv), jnp.float32),
    )(Wv, x_down)


def kernel(x, w_score, bin_prob_logits, Wv):
    Bv, Cv, Nv = x.shape
    score = jnp.einsum('bcn,c->bn', x, w_score)[:, None, :]
    s4 = (score - jnp.mean(score, axis=2, keepdims=True)) / jnp.std(
        score, axis=2, keepdims=True) + 1e-08
    s = s4.reshape(Bv, Nv)
    n = Bv * Nv
    bidx = (jnp.arange(1, _NUM_BINS) / _NUM_BINS * n).astype(jnp.int32)
    sorted_scores = jnp.sort(s.reshape(-1))[::-1]
    bvals = sorted_scores[bidx]
    binid = jnp.sum(s[:, :, None] < bvals[None, None, :], axis=2)
    counts = jnp.sum(binid[:, :, None] == jnp.arange(_NUM_BINS)[None, None, :],
                     axis=1)
    bin_prob = jnp.broadcast_to(jax.nn.softmax(bin_prob_logits)[None, :],
                                (Bv, _NUM_BINS))
    k_choose = _waterfill(_M, bin_prob, counts.astype(jnp.float32))
    rows = _build_rows(s, k_choose, bvals, _M)
    index_down = rows.reshape(Bv, 1, _M)
    idx = jnp.broadcast_to(index_down, (Bv, Cv, _M))
    x_down = jnp.take_along_axis(x, idx, axis=2)
    out = _project(Wv, x_down)
    return out, index_down


# speedup vs baseline: 195.6675x; 12.6816x over previous
import functools

import jax
import jax.numpy as jnp
from jax.experimental import pallas as pl
from jax.experimental.pallas import tpu as pltpu
from jax.experimental.pallas import tpu_sc as plsc

_NUM_BINS = 6
_M = 8192
_HALF = _M // 2
_CH = 8192


def _waterfill(total, bin_prob, max_num_points):
    Bv, num_bins = bin_prob.shape
    bp = bin_prob * max_num_points + 1e-10
    chosen = jnp.zeros_like(bp)
    for _ in range(num_bins):
        bp = bp / jnp.sum(bp, axis=1, keepdims=True)
        num_to_choose = total - jnp.sum(chosen, axis=1, keepdims=True)
        chosen = chosen + bp * num_to_choose
        chosen = jnp.where(chosen >= max_num_points, max_num_points, chosen)
        bp = bp * jnp.where(chosen >= max_num_points, 0.0, 1.0)
    chosen = chosen.astype(jnp.int32)
    deficit = total - jnp.sum(chosen, axis=1)
    arg = jnp.argmax(max_num_points.astype(jnp.int32) - chosen, axis=1)
    chosen = chosen.at[jnp.arange(Bv), arg].add(deficit.astype(chosen.dtype))
    return chosen


def _desc_key(s):
    b = jax.lax.bitcast_convert_type(s, jnp.int32)
    u = jnp.where(b < 0, ~b, b ^ jnp.int32(-2147483648)).astype(jnp.uint32)
    return ~u


def _build_rows(s, k_choose, bvals, Mv):
    Bv, Nv = s.shape
    binid = jnp.sum(s[:, :, None] < bvals[None, None, :], axis=2).astype(jnp.int32)
    onehot = binid[:, :, None] == jnp.arange(_NUM_BINS)[None, None, :]
    counts = jnp.sum(onehot, axis=1).astype(jnp.int32)
    q = s + jnp.float32(1e-08)
    offsets = jnp.cumsum(k_choose, axis=1) - k_choose
    Cj = jnp.cumsum(counts, axis=1) - counts
    iota = jnp.broadcast_to(jnp.arange(Nv, dtype=jnp.int32)[None, :], (Bv, Nv))
    keys_sorted, srt = jax.lax.sort((_desc_key(s), iota), dimension=1,
                                    is_stable=True, num_keys=1)
    u = ~keys_sorted
    hi = (u & jnp.uint32(0x80000000)) != 0
    sb = jnp.where(hi, u ^ jnp.uint32(0x80000000), ~u)
    s_sorted = jax.lax.bitcast_convert_type(sb, jnp.float32)
    q_sorted = s_sorted + jnp.float32(1e-08)
    bin_sorted = jnp.sum(
        s_sorted[:, :, None] < bvals[None, None, :], axis=2).astype(jnp.int32)

    P = jnp.sum(onehot & (q > 0)[:, :, None], axis=1).astype(jnp.int32)
    Z = jnp.sum(onehot & (q == 0)[:, :, None], axis=1).astype(jnp.int32)
    G23 = (Nv - counts) + Z
    t = jnp.minimum(k_choose, P)
    r2 = jnp.maximum(k_choose - t, 0)
    c23 = jnp.minimum(r2, G23)
    r4 = jnp.maximum(r2 - G23, 0)

    def takej(a, jj):
        r = jnp.zeros(jj.shape, a.dtype)
        for j in range(_NUM_BINS):
            r = jnp.where(jj == j, a[:, j, None], r)
        return r

    pos = jnp.arange(Nv, dtype=jnp.int32)[None, :]
    jj = bin_sorted
    w = pos - takej(Cj, jj)
    sel1 = (q_sorted > 0) & (w < takej(t, jj))
    dest1 = takej(offsets, jj) + w
    g4pos = w - takej(P, jj) - takej(Z, jj)
    sel4 = (q_sorted < 0) & (g4pos >= 0) & (g4pos < takej(r4, jj))
    dest4 = takej(offsets, jj) + takej(t, jj) + takej(c23, jj) + g4pos
    destA = jnp.where(sel1, dest1, jnp.where(sel4, dest4, Mv))

    ind23 = (binid[None, :, :] != jnp.arange(_NUM_BINS)[:, None, None]) | \
        (q == 0)[None, :, :]
    c = ind23.astype(jnp.int32)
    k = 1
    while k < Nv:
        c = c + jnp.concatenate(
            [jnp.zeros_like(c[..., :k]), c[..., :-k]], axis=-1)
        k *= 2
    pos23 = c - ind23.astype(jnp.int32)
    sel2 = ind23 & (pos23 < r2.T[:, :, None])
    dest2 = offsets.T[:, :, None] + t.T[:, :, None] + pos23
    destB = jnp.where(sel2, dest2, Mv)
    return _sc_scatter_rows(destA, srt, destB)


def _sc_rows_body(destA_hbm, srtv_hbm, destB_hbm, rows_hbm, dbuf, vbuf, rbuf):
    cid = jax.lax.axis_index("c")
    sid = jax.lax.axis_index("s")
    w = sid * 2 + cid
    b = w // 2
    hbase = (w % 2) * _HALF
    nv = _CH // 16

    zero = jnp.zeros((16,), jnp.int32)

    def zinit(i, _):
        rbuf[pl.ds(i * 16, 16)] = zero
        return 0

    jax.lax.fori_loop(0, _HALF // 16, zinit, 0)

    lane = jax.lax.iota(jnp.int32, 16)
    nchunks = destA_hbm.shape[1] // _CH

    def scatter_vec(d, v):
        m = (d >= 0) & (d < _HALF)
        dc = jnp.clip(d, 0, _HALF - 1)
        plsc.store_scatter(rbuf, [dc], v, mask=m)

    def passA(c, _):
        pltpu.sync_copy(destA_hbm.at[b, pl.ds(c * _CH, _CH)], dbuf)
        pltpu.sync_copy(srtv_hbm.at[b, pl.ds(c * _CH, _CH)], vbuf)

        def inner(i, _):
            scatter_vec(dbuf[pl.ds(i * 16, 16)] - hbase,
                        vbuf[pl.ds(i * 16, 16)])
            return 0

        jax.lax.fori_loop(0, nv, inner, 0)
        return 0

    jax.lax.fori_loop(0, nchunks, passA, 0)

    for j in range(_NUM_BINS):
        def passB(c, _, j=j):
            pltpu.sync_copy(destB_hbm.at[j, b, pl.ds(c * _CH, _CH)], dbuf)

            def inner(i, _):
                v = c * _CH + i * 16 + lane
                scatter_vec(dbuf[pl.ds(i * 16, 16)] - hbase, v)
                return 0

            jax.lax.fori_loop(0, nv, inner, 0)
            return 0

        jax.lax.fori_loop(0, nchunks, passB, 0)

    pltpu.sync_copy(rbuf, rows_hbm.at[b, pl.ds(hbase, _HALF)])


def _sc_scatter_rows(destA, srtv, destB):
    Bv = destA.shape[0]
    mesh = plsc.VectorSubcoreMesh(core_axis_name="c", subcore_axis_name="s")
    f = functools.partial(
        pl.kernel,
        out_type=jax.ShapeDtypeStruct((Bv, _M), jnp.int32),
        mesh=mesh,
        scratch_types=[
            pltpu.VMEM((_CH,), jnp.int32),
            pltpu.VMEM((_CH,), jnp.int32),
            pltpu.VMEM((_HALF,), jnp.int32),
        ],
        compiler_params=pltpu.CompilerParams(needs_layout_passes=False),
    )(_sc_rows_body)
    return f(destA, srtv, destB)


def _mm_body(wv_ref, xd_ref, o_ref):
    o_ref[0] = jax.lax.dot(
        wv_ref[...], xd_ref[0], preferred_element_type=jnp.float32)


def _project(Wv, x_down):
    Bv, Cv, Mv = x_down.shape
    BM = 2048
    return pl.pallas_call(
        _mm_body,
        grid=(Bv, Mv // BM),
        in_specs=[
            pl.BlockSpec((Cv, Cv), lambda b, m: (0, 0)),
            pl.BlockSpec((1, Cv, BM), lambda b, m: (b, 0, m)),
        ],
        out_specs=pl.BlockSpec((1, Cv, BM), lambda b, m: (b, 0, m)),
        out_shape=jax.ShapeDtypeStruct((Bv, Cv, Mv), jnp.float32),
    )(Wv, x_down)


def kernel(x, w_score, bin_prob_logits, Wv):
    Bv, Cv, Nv = x.shape
    score = jnp.einsum('bcn,c->bn', x, w_score)[:, None, :]
    s4 = (score - jnp.mean(score, axis=2, keepdims=True)) / jnp.std(
        score, axis=2, keepdims=True) + 1e-08
    s = s4.reshape(Bv, Nv)
    n = Bv * Nv
    bidx = (jnp.arange(1, _NUM_BINS) / _NUM_BINS * n).astype(jnp.int32)
    sorted_scores = jnp.sort(s.reshape(-1))[::-1]
    bvals = sorted_scores[bidx]
    binid = jnp.sum(s[:, :, None] < bvals[None, None, :], axis=2)
    counts = jnp.sum(binid[:, :, None] == jnp.arange(_NUM_BINS)[None, None, :],
                     axis=1)
    bin_prob = jnp.broadcast_to(jax.nn.softmax(bin_prob_logits)[None, :],
                                (Bv, _NUM_BINS))
    k_choose = _waterfill(_M, bin_prob, counts.astype(jnp.float32))
    rows = _build_rows(s, k_choose, bvals, _M)
    index_down = rows.reshape(Bv, 1, _M)
    idx = jnp.broadcast_to(index_down, (Bv, Cv, _M))
    x_down = jnp.take_along_axis(x, idx, axis=2)
    out = _project(Wv, x_down)
    return out, index_down
